# baseline (speedup 1.0000x reference)
import functools
import math

import jax
import jax.numpy as jnp
from jax import lax
from jax.experimental import pallas as pl
from jax.experimental.pallas import tpu as pltpu
from jax.experimental.pallas import tpu_sc as plsc

CAPACITY_FACTOR = 1.25
K_TOP = 2


def _route_body(scale_ref, x_ref, wg_ref,
                slot0_ref, slot1_ref, w0b_ref, w1b_ref, aux_ref,
                *, T, E, C, CP, BLK):
    scale = scale_ref[0, 0]
    logits = jnp.dot(x_ref[...], wg_ref[...],
                     preferred_element_type=jnp.float32) * scale

    iota_e = lax.broadcasted_iota(jnp.int32, (T, E), 1)
    m1 = jnp.max(logits, axis=1, keepdims=True)
    a1 = jnp.min(jnp.where(logits == m1, iota_e, E), axis=1, keepdims=True)
    oh0 = (iota_e == a1)
    l2 = jnp.where(oh0, jnp.float32(-1e30), logits)
    m2 = jnp.max(l2, axis=1, keepdims=True)
    a2 = jnp.min(jnp.where(l2 == m2, iota_e, E), axis=1, keepdims=True)
    oh1 = (iota_e == a2)

    z = jnp.exp(m2 - m1)
    p0 = 1.0 / (1.0 + z)
    p1 = z / (1.0 + z)

    oh0f = oh0.astype(jnp.float32)
    oh1f = oh1.astype(jnp.float32)
    ohsum = oh0f + oh1f

    nb = T // BLK
    tril = (lax.broadcasted_iota(jnp.int32, (BLK, BLK), 0)
            > lax.broadcasted_iota(jnp.int32, (BLK, BLK), 1)).astype(jnp.float32)
    carry = jnp.zeros((1, E), jnp.float32)
    pieces = []
    for b in range(nb):
        blk = ohsum[b * BLK:(b + 1) * BLK, :]
        local = jnp.dot(tril, blk, preferred_element_type=jnp.float32)
        pieces.append(local + carry)
        carry = carry + jnp.sum(blk, axis=0, keepdims=True)
    cumexcl = jnp.concatenate(pieces, axis=0)

    cnt0 = jnp.sum(oh0f * cumexcl, axis=1, keepdims=True) + 1.0
    cnt1 = jnp.sum(oh1f * cumexcl, axis=1, keepdims=True) + 1.0
    keep0 = cnt0 <= C
    keep1 = cnt1 <= C
    pos0 = cnt0.astype(jnp.int32) - 1
    pos1 = cnt1.astype(jnp.int32) - 1

    p0k = p0 * keep0.astype(jnp.float32)
    p1k = p1 * keep1.astype(jnp.float32)
    denom = p0k + p1k + 1e-9
    w0 = jnp.where(keep0, p0k / denom, 0.0)
    w1 = jnp.where(keep1, p1k / denom, 0.0)

    slot0 = jnp.where(keep0, a1 * CP + pos0, a1 * CP + C)
    slot1 = jnp.where(keep1, a2 * CP + pos1, a2 * CP + C)

    slot0_ref[...] = slot0[:, 0]
    slot1_ref[...] = slot1[:, 0]
    w0b_ref[...] = jnp.broadcast_to(w0, (T, 128))
    w1b_ref[...] = jnp.broadcast_to(w1, (T, 128))

    k0f = keep0.astype(jnp.float32)
    k1f = keep1.astype(jnp.float32)
    tokens_per_e = jnp.sum(k0f * oh0f + k1f * oh1f, axis=0, keepdims=True)
    importance = jnp.sum(w0 * oh0f + w1 * oh1f, axis=0, keepdims=True)
    tf = tokens_per_e / (jnp.sum(tokens_per_e) + 1e-9)
    imf = importance / (jnp.sum(importance) + 1e-9)
    aux_ref[0, 0] = jnp.sum(tf * imf) * E


def _route(xf, W_gate, scale, *, T, E, C, CP, interpret=False):
    body = functools.partial(_route_body, T=T, E=E, C=C, CP=CP, BLK=128)
    return pl.pallas_call(
        body,
        in_specs=[
            pl.BlockSpec(memory_space=pltpu.SMEM),
            pl.BlockSpec(memory_space=pltpu.VMEM),
            pl.BlockSpec(memory_space=pltpu.VMEM),
        ],
        out_specs=[
            pl.BlockSpec(memory_space=pltpu.VMEM),
            pl.BlockSpec(memory_space=pltpu.VMEM),
            pl.BlockSpec(memory_space=pltpu.VMEM),
            pl.BlockSpec(memory_space=pltpu.VMEM),
            pl.BlockSpec(memory_space=pltpu.SMEM),
        ],
        out_shape=[
            jax.ShapeDtypeStruct((T,), jnp.int32),
            jax.ShapeDtypeStruct((T,), jnp.int32),
            jax.ShapeDtypeStruct((T, 128), jnp.float32),
            jax.ShapeDtypeStruct((T, 128), jnp.float32),
            jax.ShapeDtypeStruct((1, 1), jnp.float32),
        ],
        interpret=interpret,
    )(scale, xf, W_gate)


def _experts_body(buf_ref, w1_ref, w2_ref, ws_ref, out_ref, acc_ref, *, nf):
    j = pl.program_id(1)

    @pl.when(j == 0)
    def _():
        acc_ref[...] = jnp.zeros_like(acc_ref)

    h = jnp.maximum(jnp.dot(buf_ref[...].astype(jnp.bfloat16),
                            w1_ref[0].astype(jnp.bfloat16),
                            preferred_element_type=jnp.float32), 0.0)
    acc_ref[...] += jnp.dot(h.astype(jnp.bfloat16),
                            w2_ref[0].astype(jnp.bfloat16),
                            preferred_element_type=jnp.float32)

    @pl.when(j == nf - 1)
    def _():
        out_ref[...] = acc_ref[...] * ws_ref[:, 0:1]


def _experts(buf, W1, W2, wslot, *, E, CP, D, DFF, FBLK=512, interpret=False):
    nf = DFF // FBLK
    body = functools.partial(_experts_body, nf=nf)
    return pl.pallas_call(
        body,
        grid=(E, nf),
        in_specs=[
            pl.BlockSpec((CP, D), lambda e, j: (e, 0)),
            pl.BlockSpec((1, D, FBLK), lambda e, j: (e, 0, j)),
            pl.BlockSpec((1, FBLK, D), lambda e, j: (e, j, 0)),
            pl.BlockSpec((CP, 128), lambda e, j: (e, 0)),
        ],
        out_specs=pl.BlockSpec((CP, D), lambda e, j: (e, 0)),
        out_shape=jax.ShapeDtypeStruct((E * CP, D), jnp.float32),
        scratch_shapes=[pltpu.VMEM((CP, D), jnp.float32)],
        compiler_params=pltpu.CompilerParams(
            dimension_semantics=("parallel", "arbitrary")),
        interpret=interpret,
    )(buf, W1, W2, wslot)


def _make_dispatch(T, D, NSLOT):
    info = plsc.get_sparse_core_info()
    NC, NS = info.num_cores, info.num_subcores
    NW = NC * NS
    per_w = T // NW
    CH = 32
    nch = per_w // CH
    mesh = plsc.VectorSubcoreMesh(core_axis_name="c", subcore_axis_name="s")

    NB = 3

    @functools.partial(
        pl.kernel, mesh=mesh,
        out_type=[
            jax.ShapeDtypeStruct((NSLOT, D), jnp.float32),
            jax.ShapeDtypeStruct((NSLOT, 128), jnp.float32),
        ],
        scratch_types=(
            [pltpu.VMEM((CH, D), jnp.float32)] * NB
            + [pltpu.VMEM((CH, 128), jnp.float32)] * 4
            + [
                pltpu.VMEM((nch, CH), jnp.int32),
                pltpu.VMEM((nch, CH), jnp.int32),
            ]
            + [pltpu.SemaphoreType.DMA] * (3 * NB + 4)
        ),
    )
    def dispatch(x_hbm, s0_hbm, s1_hbm, w0_hbm, w1_hbm,
                 buf_hbm, wslot_hbm, *rest):
        rbufs = rest[:NB]
        wv0 = rest[NB:NB + 2]
        wv1 = rest[NB + 2:NB + 4]
        i0_v, i1_v = rest[NB + 4], rest[NB + 5]
        sems = rest[NB + 6:]
        lsem = sems[:NB]
        s0sem = sems[NB:2 * NB]
        s1sem = sems[2 * NB:3 * NB]
        wsem0 = sems[3 * NB:3 * NB + 2]
        wsem1 = sems[3 * NB + 2:3 * NB + 4]

        wid = lax.axis_index("s") * NC + lax.axis_index("c")
        pltpu.sync_copy(s0_hbm.at[wid], i0_v)
        pltpu.sync_copy(s1_hbm.at[wid], i1_v)

        loads = [None] * nch
        scat = [None] * nch
        wscat = [None] * nch
        for c in range(min(NB, nch)):
            loads[c] = pltpu.async_copy(
                x_hbm.at[pl.ds(wid * per_w + c * CH, CH)],
                rbufs[c % NB], lsem[c % NB])
        for c in range(nch):
            p = c % NB
            loads[c].wait()
            scat[c] = (
                pltpu.async_copy(rbufs[p], buf_hbm.at[i0_v.at[c]], s0sem[p]),
                pltpu.async_copy(rbufs[p], buf_hbm.at[i1_v.at[c]], s1sem[p]),
            )
            wq = c % 2
            if c >= 2:
                wscat[c - 2][0].wait()
                wscat[c - 2][1].wait()
            pltpu.sync_copy(w0_hbm.at[wid, c], wv0[wq])
            pltpu.sync_copy(w1_hbm.at[wid, c], wv1[wq])
            wscat[c] = (
                pltpu.async_copy(wv0[wq], wslot_hbm.at[i0_v.at[c]],
                                 wsem0[wq]),
                pltpu.async_copy(wv1[wq], wslot_hbm.at[i1_v.at[c]],
                                 wsem1[wq]),
            )
            nl = c + 1
            if NB <= nl < nch:
                scat[nl - NB][0].wait()
                scat[nl - NB][1].wait()
                loads[nl] = pltpu.async_copy(
                    x_hbm.at[pl.ds(wid * per_w + nl * CH, CH)],
                    rbufs[nl % NB], lsem[nl % NB])
        for c in range(max(0, nch - NB), nch):
            scat[c][0].wait()
            scat[c][1].wait()
        for c in range(max(0, nch - 2), nch):
            wscat[c][0].wait()
            wscat[c][1].wait()

    return dispatch


def _make_combine(T, D, NSLOT):
    info = plsc.get_sparse_core_info()
    NC, NS = info.num_cores, info.num_subcores
    NW = NC * NS
    per_w = T // NW
    CH = 16
    nch = per_w // CH
    nvec = D // 16
    mesh = plsc.VectorSubcoreMesh(core_axis_name="c", subcore_axis_name="s")

    NB = 3

    @functools.partial(
        pl.kernel, mesh=mesh,
        out_type=jax.ShapeDtypeStruct((T, D), jnp.float32),
        scratch_types=(
            [pltpu.VMEM((CH, D), jnp.float32)] * (2 * NB)
            + [
                pltpu.VMEM((nch, CH), jnp.int32),
                pltpu.VMEM((nch, CH), jnp.int32),
            ]
            + [pltpu.SemaphoreType.DMA] * (3 * NB)
        ),
    )
    def combine(out_hbm, s0_hbm, s1_hbm, y_hbm, *rest):
        r0 = rest[:NB]
        r1 = rest[NB:2 * NB]
        i0_v, i1_v = rest[2 * NB], rest[2 * NB + 1]
        sems = rest[2 * NB + 2:]
        g0sem = sems[:NB]
        g1sem = sems[NB:2 * NB]
        ysem = sems[2 * NB:3 * NB]

        wid = lax.axis_index("s") * NC + lax.axis_index("c")
        pltpu.sync_copy(s0_hbm.at[wid], i0_v)
        pltpu.sync_copy(s1_hbm.at[wid], i1_v)
        gath = [None] * nch
        ystore = [None] * nch
        for c in range(min(NB, nch)):
            gath[c] = (
                pltpu.async_copy(out_hbm.at[i0_v.at[c]], r0[c], g0sem[c]),
                pltpu.async_copy(out_hbm.at[i1_v.at[c]], r1[c], g1sem[c]),
            )
        for c in range(nch):
            p = c % NB
            gath[c][0].wait()
            gath[c][1].wait()
            nl = c + 1
            if NB <= nl < nch:
                ystore[nl - NB].wait()
                q = nl % NB
                gath[nl] = (
                    pltpu.async_copy(out_hbm.at[i0_v.at[nl]], r0[q],
                                     g0sem[q]),
                    pltpu.async_copy(out_hbm.at[i1_v.at[nl]], r1[q],
                                     g1sem[q]),
                )
            y_v, b_v = r0[p], r1[p]

            def row(i, _, y_v=y_v, b_v=b_v):
                def col(j, __):
                    sl = pl.ds(j * 16, 16)
                    y_v[i, sl] = y_v[i, sl] + b_v[i, sl]
                    return __

                return lax.fori_loop(0, nvec, col, _, unroll=8)

            lax.fori_loop(0, CH, row, 0)
            ystore[c] = pltpu.async_copy(
                y_v, y_hbm.at[pl.ds(wid * per_w + c * CH, CH)], ysem[p])
        for c in range(max(0, nch - NB), nch):
            ystore[c].wait()

    return combine


def kernel(x, W_gate, W1, W2, k):
    B, S, D = x.shape
    E = W_gate.shape[1]
    DFF = W1.shape[2]
    T = B * S
    C = math.ceil(CAPACITY_FACTOR * T / E)
    CP = ((C + 8) + 7) // 8 * 8
    NSLOT = E * CP

    xf = x.reshape(T, D)
    scale = (jnp.asarray(k, jnp.float32) / K_TOP).reshape(1, 1)

    slot0, slot1, w0b, w1b, aux = _route(xf, W_gate, scale,
                                         T=T, E=E, C=C, CP=CP)
    NW = 32
    s0d = slot0.reshape(NW, 4, 32)
    s1d = slot1.reshape(NW, 4, 32)
    s0c = slot0.reshape(NW, 8, 16)
    s1c = slot1.reshape(NW, 8, 16)
    w0r = w0b.reshape(NW, 4, 32, 128)
    w1r = w1b.reshape(NW, 4, 32, 128)
    buf, wslot = _make_dispatch(T, D, NSLOT)(xf, s0d, s1d, w0r, w1r)
    out = _experts(buf, W1, W2, wslot, E=E, CP=CP, D=D, DFF=DFF)
    y = _make_combine(T, D, NSLOT)(out, s0c, s1c)
    return y.reshape(B, S, D), aux.reshape(())


# speedup vs baseline: 1.0704x; 1.0704x over previous
import functools
import math

import jax
import jax.numpy as jnp
from jax import lax
from jax.experimental import pallas as pl
from jax.experimental.pallas import tpu as pltpu
from jax.experimental.pallas import tpu_sc as plsc

CAPACITY_FACTOR = 1.25
K_TOP = 2


def _route_body(scale_ref, x_ref, wg_ref,
                slot0_ref, slot1_ref, w0b_ref, w1b_ref, aux_ref,
                *, T, E, C, CP, BLK):
    scale = scale_ref[0, 0]
    logits = jnp.dot(x_ref[...], wg_ref[...],
                     preferred_element_type=jnp.float32) * scale

    iota_e = lax.broadcasted_iota(jnp.int32, (T, E), 1)
    m1 = jnp.max(logits, axis=1, keepdims=True)
    a1 = jnp.min(jnp.where(logits == m1, iota_e, E), axis=1, keepdims=True)
    oh0 = (iota_e == a1)
    l2 = jnp.where(oh0, jnp.float32(-1e30), logits)
    m2 = jnp.max(l2, axis=1, keepdims=True)
    a2 = jnp.min(jnp.where(l2 == m2, iota_e, E), axis=1, keepdims=True)
    oh1 = (iota_e == a2)

    z = jnp.exp(m2 - m1)
    p0 = 1.0 / (1.0 + z)
    p1 = z / (1.0 + z)

    oh0f = oh0.astype(jnp.float32)
    oh1f = oh1.astype(jnp.float32)
    ohsum = oh0f + oh1f

    nb = T // BLK
    tril = (lax.broadcasted_iota(jnp.int32, (BLK, BLK), 0)
            > lax.broadcasted_iota(jnp.int32, (BLK, BLK), 1)).astype(jnp.float32)
    carry = jnp.zeros((1, E), jnp.float32)
    pieces = []
    for b in range(nb):
        blk = ohsum[b * BLK:(b + 1) * BLK, :]
        local = jnp.dot(tril, blk, preferred_element_type=jnp.float32)
        pieces.append(local + carry)
        carry = carry + jnp.sum(blk, axis=0, keepdims=True)
    cumexcl = jnp.concatenate(pieces, axis=0)

    cnt0 = jnp.sum(oh0f * cumexcl, axis=1, keepdims=True) + 1.0
    cnt1 = jnp.sum(oh1f * cumexcl, axis=1, keepdims=True) + 1.0
    keep0 = cnt0 <= C
    keep1 = cnt1 <= C
    pos0 = cnt0.astype(jnp.int32) - 1
    pos1 = cnt1.astype(jnp.int32) - 1

    p0k = p0 * keep0.astype(jnp.float32)
    p1k = p1 * keep1.astype(jnp.float32)
    denom = p0k + p1k + 1e-9
    w0 = jnp.where(keep0, p0k / denom, 0.0)
    w1 = jnp.where(keep1, p1k / denom, 0.0)

    slot0 = jnp.where(keep0, a1 * CP + pos0, a1 * CP + C)
    slot1 = jnp.where(keep1, a2 * CP + pos1, a2 * CP + C)

    slot0_ref[...] = slot0[:, 0]
    slot1_ref[...] = slot1[:, 0]
    w0b_ref[...] = jnp.broadcast_to(w0, (T, 128))
    w1b_ref[...] = jnp.broadcast_to(w1, (T, 128))

    k0f = keep0.astype(jnp.float32)
    k1f = keep1.astype(jnp.float32)
    tokens_per_e = jnp.sum(k0f * oh0f + k1f * oh1f, axis=0, keepdims=True)
    importance = jnp.sum(w0 * oh0f + w1 * oh1f, axis=0, keepdims=True)
    tf = tokens_per_e / (jnp.sum(tokens_per_e) + 1e-9)
    imf = importance / (jnp.sum(importance) + 1e-9)
    aux_ref[0, 0] = jnp.sum(tf * imf) * E


def _route(xf, W_gate, scale, *, T, E, C, CP, interpret=False):
    body = functools.partial(_route_body, T=T, E=E, C=C, CP=CP, BLK=128)
    return pl.pallas_call(
        body,
        in_specs=[
            pl.BlockSpec(memory_space=pltpu.SMEM),
            pl.BlockSpec(memory_space=pltpu.VMEM),
            pl.BlockSpec(memory_space=pltpu.VMEM),
        ],
        out_specs=[
            pl.BlockSpec(memory_space=pltpu.VMEM),
            pl.BlockSpec(memory_space=pltpu.VMEM),
            pl.BlockSpec(memory_space=pltpu.VMEM),
            pl.BlockSpec(memory_space=pltpu.VMEM),
            pl.BlockSpec(memory_space=pltpu.SMEM),
        ],
        out_shape=[
            jax.ShapeDtypeStruct((T,), jnp.int32),
            jax.ShapeDtypeStruct((T,), jnp.int32),
            jax.ShapeDtypeStruct((T, 128), jnp.float32),
            jax.ShapeDtypeStruct((T, 128), jnp.float32),
            jax.ShapeDtypeStruct((1, 1), jnp.float32),
        ],
        interpret=interpret,
    )(scale, xf, W_gate)


def _experts_body(buf_ref, w1_ref, w2_ref, ws_ref, out_ref, acc_ref, *, nf):
    j = pl.program_id(1)

    @pl.when(j == 0)
    def _():
        acc_ref[...] = jnp.zeros_like(acc_ref)

    h = jnp.maximum(jnp.dot(buf_ref[...], w1_ref[0],
                            preferred_element_type=jnp.float32), 0.0)
    acc_ref[...] += jnp.dot(h, w2_ref[0],
                            preferred_element_type=jnp.float32)

    @pl.when(j == nf - 1)
    def _():
        out_ref[...] = acc_ref[...] * ws_ref[:, 0:1]


def _experts(buf, W1, W2, wslot, *, E, CP, D, DFF, FBLK=1024, interpret=False):
    nf = DFF // FBLK
    body = functools.partial(_experts_body, nf=nf)
    return pl.pallas_call(
        body,
        grid=(E, nf),
        in_specs=[
            pl.BlockSpec((CP, D), lambda e, j: (e, 0)),
            pl.BlockSpec((1, D, FBLK), lambda e, j: (e, 0, j)),
            pl.BlockSpec((1, FBLK, D), lambda e, j: (e, j, 0)),
            pl.BlockSpec((CP, 128), lambda e, j: (e, 0)),
        ],
        out_specs=pl.BlockSpec((CP, D), lambda e, j: (e, 0)),
        out_shape=jax.ShapeDtypeStruct((E * CP, D), jnp.float32),
        scratch_shapes=[pltpu.VMEM((CP, D), jnp.float32)],
        compiler_params=pltpu.CompilerParams(
            dimension_semantics=("parallel", "arbitrary")),
        interpret=interpret,
    )(buf, W1, W2, wslot)


def _make_dispatch(T, D, NSLOT):
    info = plsc.get_sparse_core_info()
    NC, NS = info.num_cores, info.num_subcores
    NW = NC * NS
    per_w = T // NW
    CH = 32
    nch = per_w // CH
    mesh = plsc.VectorSubcoreMesh(core_axis_name="c", subcore_axis_name="s")

    NB = 3

    @functools.partial(
        pl.kernel, mesh=mesh,
        out_type=[
            jax.ShapeDtypeStruct((NSLOT, D), jnp.float32),
            jax.ShapeDtypeStruct((NSLOT, 128), jnp.float32),
        ],
        scratch_types=(
            [pltpu.VMEM((CH, D), jnp.float32)] * NB
            + [pltpu.VMEM((CH, 128), jnp.float32)] * 4
            + [
                pltpu.VMEM((nch, CH), jnp.int32),
                pltpu.VMEM((nch, CH), jnp.int32),
            ]
            + [pltpu.SemaphoreType.DMA] * (3 * NB + 4)
        ),
    )
    def dispatch(x_hbm, s0_hbm, s1_hbm, w0_hbm, w1_hbm,
                 buf_hbm, wslot_hbm, *rest):
        rbufs = rest[:NB]
        wv0 = rest[NB:NB + 2]
        wv1 = rest[NB + 2:NB + 4]
        i0_v, i1_v = rest[NB + 4], rest[NB + 5]
        sems = rest[NB + 6:]
        lsem = sems[:NB]
        s0sem = sems[NB:2 * NB]
        s1sem = sems[2 * NB:3 * NB]
        wsem0 = sems[3 * NB:3 * NB + 2]
        wsem1 = sems[3 * NB + 2:3 * NB + 4]

        wid = lax.axis_index("s") * NC + lax.axis_index("c")
        pltpu.sync_copy(s0_hbm.at[wid], i0_v)
        pltpu.sync_copy(s1_hbm.at[wid], i1_v)

        loads = [None] * nch
        scat = [None] * nch
        wscat = [None] * nch
        for c in range(min(NB, nch)):
            loads[c] = pltpu.async_copy(
                x_hbm.at[pl.ds(wid * per_w + c * CH, CH)],
                rbufs[c % NB], lsem[c % NB])
        for c in range(nch):
            p = c % NB
            loads[c].wait()
            scat[c] = (
                pltpu.async_copy(rbufs[p], buf_hbm.at[i0_v.at[c]], s0sem[p]),
                pltpu.async_copy(rbufs[p], buf_hbm.at[i1_v.at[c]], s1sem[p]),
            )
            wq = c % 2
            if c >= 2:
                wscat[c - 2][0].wait()
                wscat[c - 2][1].wait()
            pltpu.sync_copy(w0_hbm.at[wid, c], wv0[wq])
            pltpu.sync_copy(w1_hbm.at[wid, c], wv1[wq])
            wscat[c] = (
                pltpu.async_copy(wv0[wq], wslot_hbm.at[i0_v.at[c]],
                                 wsem0[wq]),
                pltpu.async_copy(wv1[wq], wslot_hbm.at[i1_v.at[c]],
                                 wsem1[wq]),
            )
            nl = c + 1
            if NB <= nl < nch:
                scat[nl - NB][0].wait()
                scat[nl - NB][1].wait()
                loads[nl] = pltpu.async_copy(
                    x_hbm.at[pl.ds(wid * per_w + nl * CH, CH)],
                    rbufs[nl % NB], lsem[nl % NB])
        for c in range(max(0, nch - NB), nch):
            scat[c][0].wait()
            scat[c][1].wait()
        for c in range(max(0, nch - 2), nch):
            wscat[c][0].wait()
            wscat[c][1].wait()

    return dispatch


def _make_combine(T, D, NSLOT):
    info = plsc.get_sparse_core_info()
    NC, NS = info.num_cores, info.num_subcores
    NW = NC * NS
    per_w = T // NW
    CH = 16
    nch = per_w // CH
    nvec = D // 16
    mesh = plsc.VectorSubcoreMesh(core_axis_name="c", subcore_axis_name="s")

    NB = 3

    @functools.partial(
        pl.kernel, mesh=mesh,
        out_type=jax.ShapeDtypeStruct((T, D), jnp.float32),
        scratch_types=(
            [pltpu.VMEM((CH, D), jnp.float32)] * (2 * NB)
            + [
                pltpu.VMEM((nch, CH), jnp.int32),
                pltpu.VMEM((nch, CH), jnp.int32),
            ]
            + [pltpu.SemaphoreType.DMA] * (3 * NB)
        ),
    )
    def combine(out_hbm, s0_hbm, s1_hbm, y_hbm, *rest):
        r0 = rest[:NB]
        r1 = rest[NB:2 * NB]
        i0_v, i1_v = rest[2 * NB], rest[2 * NB + 1]
        sems = rest[2 * NB + 2:]
        g0sem = sems[:NB]
        g1sem = sems[NB:2 * NB]
        ysem = sems[2 * NB:3 * NB]

        wid = lax.axis_index("s") * NC + lax.axis_index("c")
        pltpu.sync_copy(s0_hbm.at[wid], i0_v)
        pltpu.sync_copy(s1_hbm.at[wid], i1_v)
        gath = [None] * nch
        ystore = [None] * nch
        for c in range(min(NB, nch)):
            gath[c] = (
                pltpu.async_copy(out_hbm.at[i0_v.at[c]], r0[c], g0sem[c]),
                pltpu.async_copy(out_hbm.at[i1_v.at[c]], r1[c], g1sem[c]),
            )
        for c in range(nch):
            p = c % NB
            gath[c][0].wait()
            gath[c][1].wait()
            nl = c + 1
            if NB <= nl < nch:
                ystore[nl - NB].wait()
                q = nl % NB
                gath[nl] = (
                    pltpu.async_copy(out_hbm.at[i0_v.at[nl]], r0[q],
                                     g0sem[q]),
                    pltpu.async_copy(out_hbm.at[i1_v.at[nl]], r1[q],
                                     g1sem[q]),
                )
            y_v, b_v = r0[p], r1[p]

            def row(i, _, y_v=y_v, b_v=b_v):
                def col(j, __):
                    sl = pl.ds(j * 16, 16)
                    y_v[i, sl] = y_v[i, sl] + b_v[i, sl]
                    return __

                return lax.fori_loop(0, nvec, col, _, unroll=16)

            lax.fori_loop(0, CH, row, 0)
            ystore[c] = pltpu.async_copy(
                y_v, y_hbm.at[pl.ds(wid * per_w + c * CH, CH)], ysem[p])
        for c in range(max(0, nch - NB), nch):
            ystore[c].wait()

    return combine


def kernel(x, W_gate, W1, W2, k):
    B, S, D = x.shape
    E = W_gate.shape[1]
    DFF = W1.shape[2]
    T = B * S
    C = math.ceil(CAPACITY_FACTOR * T / E)
    CP = ((C + 8) + 7) // 8 * 8
    NSLOT = E * CP

    xf = x.reshape(T, D)
    scale = (jnp.asarray(k, jnp.float32) / K_TOP).reshape(1, 1)

    slot0, slot1, w0b, w1b, aux = _route(xf, W_gate, scale,
                                         T=T, E=E, C=C, CP=CP)
    NW = 32
    s0d = slot0.reshape(NW, 4, 32)
    s1d = slot1.reshape(NW, 4, 32)
    s0c = slot0.reshape(NW, 8, 16)
    s1c = slot1.reshape(NW, 8, 16)
    w0r = w0b.reshape(NW, 4, 32, 128)
    w1r = w1b.reshape(NW, 4, 32, 128)
    buf, wslot = _make_dispatch(T, D, NSLOT)(xf, s0d, s1d, w0r, w1r)
    out = _experts(buf, W1, W2, wslot, E=E, CP=CP, D=D, DFF=DFF)
    y = _make_combine(T, D, NSLOT)(out, s0c, s1c)
    return y.reshape(B, S, D), aux.reshape(())
